# Initial kernel scaffold; baseline (speedup 1.0000x reference)
import jax
import jax.numpy as jnp
from jax.experimental import pallas as pl


def kernel(x, y, senders, receivers, rel_pos, window_support, a, W1, b1, bn1_scale, bn1_offset, W2, b2, bn2_scale, bn2_offset):
    raise NotImplementedError("write your pallas kernel here")


# speedup vs baseline: 1.8571x; 1.8571x over previous
import functools

import jax
import jax.numpy as jnp
from jax import lax
from jax.experimental import pallas as pl
from jax.experimental.pallas import tpu as pltpu
from jax.experimental.pallas import tpu_sc as plsc

_N = 10000
_E = 160000
_K = 4

_TILES = 32
_B = 32
_CH = 10
_GC = 16
_G = _CH * _GC
_EPAD = _TILES * _G * _B
_NA = 10240
_NPT = _NA // 16


def _prep_body(ws_ref, a_ref, px_ref, py_ref, snd_ref, w_ref, g_ref):
    ws = ws_ref[0, 0]
    a = a_ref[0, 0]
    dx = px_ref[...] / ws
    dy = py_ref[...] / ws
    r2 = dx * dx + dy * dy
    win = jnp.power(jnp.maximum(1.0 - r2, 0.0), a)
    ux = (jnp.clip(dx, -1.0, 1.0) + 1.0) * (0.5 * (_K - 1))
    uy = (jnp.clip(dy, -1.0, 1.0) + 1.0) * (0.5 * (_K - 1))
    ix = jnp.clip(jnp.floor(ux).astype(jnp.int32), 0, _K - 2)
    iy = jnp.clip(jnp.floor(uy).astype(jnp.int32), 0, _K - 2)
    fx = ux - ix.astype(jnp.float32)
    fy = uy - iy.astype(jnp.float32)
    base = snd_ref[...] * (_K * _K) + iy * _K + ix
    c = 0
    for ddy in (0, 1):
        cy = fy if ddy == 1 else 1.0 - fy
        for ddx in (0, 1):
            cx = fx if ddx == 1 else 1.0 - fx
            w_ref[c] = win * cy * cx
            g_ref[c] = base + ddy * _K + ddx
            c += 1


def _edge_prep(rel_pos, senders, ws, a):
    er = _E // 128
    py = rel_pos[:, 0].reshape(er, 128)
    px = rel_pos[:, 1].reshape(er, 128)
    snd = senders.astype(jnp.int32).reshape(er, 128)
    ws_s = jnp.asarray(ws, jnp.float32).reshape(1, 1)
    a_s = jnp.asarray(a, jnp.float32).reshape(1, 1)
    w4, g4 = pl.pallas_call(
        _prep_body,
        in_specs=[
            pl.BlockSpec(memory_space=pltpu.SMEM),
            pl.BlockSpec(memory_space=pltpu.SMEM),
            pl.BlockSpec((er, 128), lambda: (0, 0)),
            pl.BlockSpec((er, 128), lambda: (0, 0)),
            pl.BlockSpec((er, 128), lambda: (0, 0)),
        ],
        out_specs=[
            pl.BlockSpec((4, er, 128), lambda: (0, 0, 0)),
            pl.BlockSpec((4, er, 128), lambda: (0, 0, 0)),
        ],
        out_shape=[
            jax.ShapeDtypeStruct((4, er, 128), jnp.float32),
            jax.ShapeDtypeStruct((4, er, 128), jnp.int32),
        ],
    )(ws_s, a_s, px, py, snd)
    wE = w4.transpose(1, 2, 0).reshape(_E * 4)
    gE = g4.transpose(1, 2, 0).reshape(_E * 4)
    wE = jnp.pad(wE, (0, _EPAD * 4 - _E * 4)).reshape(_TILES, _CH, _GC, 4 * _B)
    gE = jnp.pad(gE, (0, _EPAD * 4 - _E * 4)).reshape(_TILES, _CH, _GC, 4 * _B)
    return wE, gE


def _mm_body(f_ref, w_ref, o_ref):
    o_ref[...] = jnp.dot(f_ref[...], w_ref[...],
                         preferred_element_type=jnp.float32)


def _tap_transform(feat, w2d):
    np_, cin = feat.shape
    o2 = w2d.shape[1]
    bn = 512
    h = pl.pallas_call(
        _mm_body,
        grid=(np_ // bn,),
        in_specs=[pl.BlockSpec((bn, cin), lambda i: (i, 0)),
                  pl.BlockSpec((cin, o2), lambda i: (0, 0))],
        out_specs=pl.BlockSpec((bn, o2), lambda i: (i, 0)),
        out_shape=jax.ShapeDtypeStruct((np_, o2), jnp.float32),
    )(feat, w2d)
    return h.reshape(np_ * _K * _K, o2 // (_K * _K))


def _bn_relu_body(p_ref, sc_ref, of_ref, o_ref):
    s = p_ref[0] + p_ref[1]
    m = jnp.mean(s, axis=0, keepdims=True)
    v = jnp.mean(s * s, axis=0, keepdims=True) - m * m
    z = sc_ref[...] * (s - m) * lax.rsqrt(v + 1e-5) + of_ref[...]
    o_ref[...] = jnp.maximum(z, 0.0)


def _bn_relu(p, scale, offset):
    return pl.pallas_call(
        _bn_relu_body,
        in_specs=[pl.BlockSpec((2, _N, 128), lambda: (0, 0, 0)),
                  pl.BlockSpec((1, 128), lambda: (0, 0)),
                  pl.BlockSpec((1, 128), lambda: (0, 0))],
        out_specs=pl.BlockSpec((_N, 128), lambda: (0, 0)),
        out_shape=jax.ShapeDtypeStruct((_N, 128), jnp.float32),
    )(p, scale.reshape(1, 128), offset.reshape(1, 128))


def _final_body(p_ref, sc_ref, of_ref, x_ref, y_ref, o_ref):
    s = p_ref[0] + p_ref[1]
    m = jnp.mean(s, axis=0, keepdims=True)
    v = jnp.mean(s * s, axis=0, keepdims=True) - m * m
    z = sc_ref[...] * (s - m) * lax.rsqrt(v + 1e-5) + of_ref[...]
    wei = jax.nn.sigmoid(z)
    o_ref[...] = 2.0 * x_ref[...] * wei + 2.0 * y_ref[...] * (1.0 - wei)


def _finalize(p, scale, offset, x, y):
    return pl.pallas_call(
        _final_body,
        in_specs=[pl.BlockSpec((2, _N, 128), lambda: (0, 0, 0)),
                  pl.BlockSpec((1, 128), lambda: (0, 0)),
                  pl.BlockSpec((1, 128), lambda: (0, 0)),
                  pl.BlockSpec((_N, 128), lambda: (0, 0)),
                  pl.BlockSpec((_N, 128), lambda: (0, 0))],
        out_specs=pl.BlockSpec((_N, 128), lambda: (0, 0)),
        out_shape=jax.ShapeDtypeStruct((_N, 128), jnp.float32),
    )(p, scale.reshape(1, 128), offset.reshape(1, 128), x, y)


def _sc_scatter_body(gidx_hbm, w_hbm, recv_hbm, table_hbm, out_hbm,
                     gidx_v, w_v, recv_v, gath_v, comb_v, acc_sh, sem):
    c = lax.axis_index("c")
    s = lax.axis_index("s")
    wid = s * 2 + c

    zvec = jnp.zeros((16,), jnp.float32)
    for i in range(_B):
        for j in range(8):
            comb_v[i, pl.ds(j * 16, 16)] = zvec

    def zbody(i, carry):
        pltpu.sync_copy(comb_v, acc_sh.at[pl.ds(s * _NPT + i * _B, _B)])
        return carry
    lax.fori_loop(0, _NPT // _B, zbody, 0)
    plsc.subcore_barrier()

    def chunk_body(ch, carry):
        pltpu.sync_copy(gidx_hbm.at[wid, ch], gidx_v)
        pltpu.sync_copy(w_hbm.at[wid, ch], w_v)
        pltpu.sync_copy(recv_hbm.at[wid, ch], recv_v)

        def body(g, gcarry):
            pltpu.async_copy(table_hbm.at[gidx_v.at[g]], gath_v, sem).wait()

            def qbody(q, qcarry):
                wq = w_v[g, pl.ds(16 * q, 16)]
                for t in range(4):
                    e = 4 * q + t
                    r = 16 * q + 4 * t
                    w0, w1, w2, w3 = wq[4 * t], wq[4 * t + 1], wq[4 * t + 2], wq[4 * t + 3]
                    for j in range(8):
                        sl = pl.ds(j * 16, 16)
                        v = (w0 * gath_v[r, sl] + w1 * gath_v[r + 1, sl]
                             + w2 * gath_v[r + 2, sl] + w3 * gath_v[r + 3, sl])
                        comb_v[e, sl] = v
                return qcarry
            lax.fori_loop(0, _B // 4, qbody, 0)
            pltpu.sync_copy(comb_v, acc_sh.at[recv_v.at[g]], add=True)
            return gcarry
        lax.fori_loop(0, _GC, body, 0)
        return carry
    lax.fori_loop(0, _CH, chunk_body, 0)

    plsc.subcore_barrier()
    pltpu.sync_copy(acc_sh.at[pl.ds(s * _NPT, _NPT)],
                    out_hbm.at[c, pl.ds(s * _NPT, _NPT)])


def _sc_scatter(gidx, w, recv, table):
    mesh = plsc.VectorSubcoreMesh(core_axis_name="c", subcore_axis_name="s")
    fn = functools.partial(
        pl.kernel,
        mesh=mesh,
        out_type=jax.ShapeDtypeStruct((2, _NA, 128), jnp.float32),
        scratch_types=[
            pltpu.VMEM((_GC, 4 * _B), jnp.int32),
            pltpu.VMEM((_GC, 4 * _B), jnp.float32),
            pltpu.VMEM((_GC, _B), jnp.int32),
            pltpu.VMEM((4 * _B, 128), jnp.float32),
            pltpu.VMEM((_B, 128), jnp.float32),
            pltpu.VMEM_SHARED((_NA, 128), jnp.float32),
            pltpu.SemaphoreType.DMA,
        ],
    )(_sc_scatter_body)
    return fn(gidx, w, recv, table)


def kernel(x, y, senders, receivers, rel_pos, window_support, a,
           W1, b1, bn1_scale, bn1_offset, W2, b2, bn2_scale, bn2_offset):
    n = x.shape[0]
    kk = _K * _K

    wE, gE = _edge_prep(rel_pos, senders, window_support, a)
    recv = jnp.pad(receivers.astype(jnp.int32),
                   (0, _EPAD - _E)).reshape(_TILES, _CH, _GC, _B)

    xa = jnp.concatenate([x, y], axis=-1)
    np1 = 10240
    xa_p = jnp.pad(xa, ((0, np1 - n), (0, 0)))
    w1_2d = W1.reshape(kk, 2 * x.shape[1], -1).transpose(1, 0, 2).reshape(
        2 * x.shape[1], kk * W1.shape[-1])
    h1 = _tap_transform(xa_p, w1_2d)
    p1 = _sc_scatter(gE, wE, recv, h1)[:, :n, :]
    xl = _bn_relu(p1, bn1_scale, bn1_offset)

    xl_p = jnp.pad(xl, ((0, np1 - n), (0, 0)))
    w2_2d = W2.reshape(kk, W2.shape[2], -1).transpose(1, 0, 2).reshape(
        W2.shape[2], kk * W2.shape[-1])
    h2 = _tap_transform(xl_p, w2_2d)
    p2 = _sc_scatter(gE, wE, recv, h2)[:, :n, :]
    return _finalize(p2, bn2_scale, bn2_offset, x, y)


# speedup vs baseline: 2.3586x; 1.2700x over previous
import functools

import jax
import jax.numpy as jnp
from jax import lax
from jax.experimental import pallas as pl
from jax.experimental.pallas import tpu as pltpu
from jax.experimental.pallas import tpu_sc as plsc

_N = 10000
_E = 160000
_K = 4

_TILES = 32
_B = 32
_CH = 10
_GC = 16
_G = _CH * _GC
_EPAD = _TILES * _G * _B
_NA = 10240
_NPT = _NA // 16


def _prep_body(ws_ref, a_ref, px_ref, py_ref, snd_ref, w_ref, g_ref):
    ws = ws_ref[0, 0]
    a = a_ref[0, 0]
    dx = px_ref[...] / ws
    dy = py_ref[...] / ws
    r2 = dx * dx + dy * dy
    win = jnp.power(jnp.maximum(1.0 - r2, 0.0), a)
    ux = (jnp.clip(dx, -1.0, 1.0) + 1.0) * (0.5 * (_K - 1))
    uy = (jnp.clip(dy, -1.0, 1.0) + 1.0) * (0.5 * (_K - 1))
    ix = jnp.clip(jnp.floor(ux).astype(jnp.int32), 0, _K - 2)
    iy = jnp.clip(jnp.floor(uy).astype(jnp.int32), 0, _K - 2)
    fx = ux - ix.astype(jnp.float32)
    fy = uy - iy.astype(jnp.float32)
    base = snd_ref[...] * (_K * _K) + iy * _K + ix
    c = 0
    for ddy in (0, 1):
        cy = fy if ddy == 1 else 1.0 - fy
        for ddx in (0, 1):
            cx = fx if ddx == 1 else 1.0 - fx
            w_ref[c] = win * cy * cx
            g_ref[c] = base + ddy * _K + ddx
            c += 1


def _edge_prep(rel_pos, senders, ws, a):
    er = _E // 128
    py = rel_pos[:, 0].reshape(er, 128)
    px = rel_pos[:, 1].reshape(er, 128)
    snd = senders.astype(jnp.int32).reshape(er, 128)
    ws_s = jnp.asarray(ws, jnp.float32).reshape(1, 1)
    a_s = jnp.asarray(a, jnp.float32).reshape(1, 1)
    w4, g4 = pl.pallas_call(
        _prep_body,
        in_specs=[
            pl.BlockSpec(memory_space=pltpu.SMEM),
            pl.BlockSpec(memory_space=pltpu.SMEM),
            pl.BlockSpec((er, 128), lambda: (0, 0)),
            pl.BlockSpec((er, 128), lambda: (0, 0)),
            pl.BlockSpec((er, 128), lambda: (0, 0)),
        ],
        out_specs=[
            pl.BlockSpec((4, er, 128), lambda: (0, 0, 0)),
            pl.BlockSpec((4, er, 128), lambda: (0, 0, 0)),
        ],
        out_shape=[
            jax.ShapeDtypeStruct((4, er, 128), jnp.float32),
            jax.ShapeDtypeStruct((4, er, 128), jnp.int32),
        ],
    )(ws_s, a_s, px, py, snd)
    wE = w4.transpose(1, 2, 0).reshape(_E * 4)
    gE = g4.transpose(1, 2, 0).reshape(_E * 4)
    wE = jnp.pad(wE, (0, _EPAD * 4 - _E * 4)).reshape(_TILES, _CH, _GC, 4 * _B)
    gE = jnp.pad(gE, (0, _EPAD * 4 - _E * 4)).reshape(_TILES, _CH, _GC, 4 * _B)
    return wE, gE


def _mm_body(f_ref, w_ref, o_ref):
    o_ref[...] = jnp.dot(f_ref[...], w_ref[...],
                         preferred_element_type=jnp.float32)


def _tap_transform(feat, w2d):
    np_, cin = feat.shape
    o2 = w2d.shape[1]
    bn = 512
    h = pl.pallas_call(
        _mm_body,
        grid=(np_ // bn,),
        in_specs=[pl.BlockSpec((bn, cin), lambda i: (i, 0)),
                  pl.BlockSpec((cin, o2), lambda i: (0, 0))],
        out_specs=pl.BlockSpec((bn, o2), lambda i: (i, 0)),
        out_shape=jax.ShapeDtypeStruct((np_, o2), jnp.float32),
    )(feat, w2d)
    return h.reshape(np_ * _K * _K, o2 // (_K * _K))


def _bn_relu_body(p_ref, sc_ref, of_ref, o_ref):
    s = p_ref[0] + p_ref[1]
    m = jnp.mean(s, axis=0, keepdims=True)
    v = jnp.mean(s * s, axis=0, keepdims=True) - m * m
    z = sc_ref[...] * (s - m) * lax.rsqrt(v + 1e-5) + of_ref[...]
    o_ref[...] = jnp.maximum(z, 0.0)


def _bn_relu(p, scale, offset):
    return pl.pallas_call(
        _bn_relu_body,
        in_specs=[pl.BlockSpec((2, _N, 128), lambda: (0, 0, 0)),
                  pl.BlockSpec((1, 128), lambda: (0, 0)),
                  pl.BlockSpec((1, 128), lambda: (0, 0))],
        out_specs=pl.BlockSpec((_N, 128), lambda: (0, 0)),
        out_shape=jax.ShapeDtypeStruct((_N, 128), jnp.float32),
    )(p, scale.reshape(1, 128), offset.reshape(1, 128))


def _final_body(p_ref, sc_ref, of_ref, x_ref, y_ref, o_ref):
    s = p_ref[0] + p_ref[1]
    m = jnp.mean(s, axis=0, keepdims=True)
    v = jnp.mean(s * s, axis=0, keepdims=True) - m * m
    z = sc_ref[...] * (s - m) * lax.rsqrt(v + 1e-5) + of_ref[...]
    wei = jax.nn.sigmoid(z)
    o_ref[...] = 2.0 * x_ref[...] * wei + 2.0 * y_ref[...] * (1.0 - wei)


def _finalize(p, scale, offset, x, y):
    return pl.pallas_call(
        _final_body,
        in_specs=[pl.BlockSpec((2, _N, 128), lambda: (0, 0, 0)),
                  pl.BlockSpec((1, 128), lambda: (0, 0)),
                  pl.BlockSpec((1, 128), lambda: (0, 0)),
                  pl.BlockSpec((_N, 128), lambda: (0, 0)),
                  pl.BlockSpec((_N, 128), lambda: (0, 0))],
        out_specs=pl.BlockSpec((_N, 128), lambda: (0, 0)),
        out_shape=jax.ShapeDtypeStruct((_N, 128), jnp.float32),
    )(p, scale.reshape(1, 128), offset.reshape(1, 128), x, y)


def _sc_scatter_body(gidx_hbm, w_hbm, recv_hbm, table_hbm, out_hbm,
                     gidx_v, w_v, recv_v, gath_a, gath_b, comb_a, comb_b,
                     acc_sh, sem_m, sem_ga, sem_gb, sem_sa, sem_sb):
    c = lax.axis_index("c")
    s = lax.axis_index("s")
    wid = s * 2 + c

    zvec = jnp.zeros((16,), jnp.float32)
    for i in range(_B):
        for j in range(8):
            comb_a[i, pl.ds(j * 16, 16)] = zvec

    def zbody(i, carry):
        pltpu.sync_copy(comb_a, acc_sh.at[pl.ds(s * _NPT + i * _B, _B)])
        return carry
    lax.fori_loop(0, _NPT // _B, zbody, 0)
    plsc.subcore_barrier()

    def compute(g, gath_v, comb_v):
        def qbody(q, qcarry):
            wq = w_v[g, pl.ds(16 * q, 16)]
            for t in range(4):
                e = 4 * q + t
                r = 16 * q + 4 * t
                w0, w1, w2, w3 = wq[4 * t], wq[4 * t + 1], wq[4 * t + 2], wq[4 * t + 3]
                for j in range(8):
                    sl = pl.ds(j * 16, 16)
                    v = (w0 * gath_v[r, sl] + w1 * gath_v[r + 1, sl]
                         + w2 * gath_v[r + 2, sl] + w3 * gath_v[r + 3, sl])
                    comb_v[e, sl] = v
            return qcarry
        lax.fori_loop(0, _B // 4, qbody, 0)

    def drain_gather(gath_v, sem):
        pltpu.make_async_copy(table_hbm.at[pl.ds(0, 4 * _B)], gath_v, sem).wait()

    def drain_scatter(comb_v, sem):
        pltpu.make_async_copy(table_hbm.at[pl.ds(0, _B)], comb_v, sem).wait()

    def chunk_body(ch, carry):
        pltpu.async_copy(gidx_hbm.at[wid, ch], gidx_v, sem_m)
        pltpu.async_copy(w_hbm.at[wid, ch], w_v, sem_m)
        pltpu.async_copy(recv_hbm.at[wid, ch], recv_v, sem_m)
        pltpu.make_async_copy(gidx_hbm.at[wid, ch], gidx_v, sem_m).wait()
        pltpu.make_async_copy(w_hbm.at[wid, ch], w_v, sem_m).wait()
        pltpu.make_async_copy(recv_hbm.at[wid, ch], recv_v, sem_m).wait()

        pltpu.async_copy(table_hbm.at[gidx_v.at[0]], gath_a, sem_ga)
        pltpu.async_copy(table_hbm.at[gidx_v.at[1]], gath_b, sem_gb)

        def pair_body(p, pcarry):
            g0 = 2 * p
            g1 = 2 * p + 1
            drain_gather(gath_a, sem_ga)

            @pl.when(p > 0)
            def _():
                drain_scatter(comb_a, sem_sa)
            compute(g0, gath_a, comb_a)

            @pl.when(p < _GC // 2 - 1)
            def _():
                pltpu.async_copy(table_hbm.at[gidx_v.at[g0 + 2]], gath_a, sem_ga)
            pltpu.async_copy(comb_a, acc_sh.at[recv_v.at[g0]], sem_sa, add=True)

            drain_gather(gath_b, sem_gb)

            @pl.when(p > 0)
            def _():
                drain_scatter(comb_b, sem_sb)
            compute(g1, gath_b, comb_b)

            @pl.when(p < _GC // 2 - 1)
            def _():
                pltpu.async_copy(table_hbm.at[gidx_v.at[g1 + 2]], gath_b, sem_gb)
            pltpu.async_copy(comb_b, acc_sh.at[recv_v.at[g1]], sem_sb, add=True)
            return pcarry
        lax.fori_loop(0, _GC // 2, pair_body, 0)
        drain_scatter(comb_a, sem_sa)
        drain_scatter(comb_b, sem_sb)
        return carry
    lax.fori_loop(0, _CH, chunk_body, 0)

    plsc.subcore_barrier()
    pltpu.sync_copy(acc_sh.at[pl.ds(s * _NPT, _NPT)],
                    out_hbm.at[c, pl.ds(s * _NPT, _NPT)])


def _sc_scatter(gidx, w, recv, table):
    mesh = plsc.VectorSubcoreMesh(core_axis_name="c", subcore_axis_name="s")
    fn = functools.partial(
        pl.kernel,
        mesh=mesh,
        out_type=jax.ShapeDtypeStruct((2, _NA, 128), jnp.float32),
        scratch_types=[
            pltpu.VMEM((_GC, 4 * _B), jnp.int32),
            pltpu.VMEM((_GC, 4 * _B), jnp.float32),
            pltpu.VMEM((_GC, _B), jnp.int32),
            pltpu.VMEM((4 * _B, 128), jnp.float32),
            pltpu.VMEM((4 * _B, 128), jnp.float32),
            pltpu.VMEM((_B, 128), jnp.float32),
            pltpu.VMEM((_B, 128), jnp.float32),
            pltpu.VMEM_SHARED((_NA, 128), jnp.float32),
            pltpu.SemaphoreType.DMA,
            pltpu.SemaphoreType.DMA,
            pltpu.SemaphoreType.DMA,
            pltpu.SemaphoreType.DMA,
            pltpu.SemaphoreType.DMA,
        ],
    )(_sc_scatter_body)
    return fn(gidx, w, recv, table)


def kernel(x, y, senders, receivers, rel_pos, window_support, a,
           W1, b1, bn1_scale, bn1_offset, W2, b2, bn2_scale, bn2_offset):
    n = x.shape[0]
    kk = _K * _K

    wE, gE = _edge_prep(rel_pos, senders, window_support, a)
    recv = jnp.pad(receivers.astype(jnp.int32),
                   (0, _EPAD - _E)).reshape(_TILES, _CH, _GC, _B)

    xa = jnp.concatenate([x, y], axis=-1)
    np1 = 10240
    xa_p = jnp.pad(xa, ((0, np1 - n), (0, 0)))
    w1_2d = W1.reshape(kk, 2 * x.shape[1], -1).transpose(1, 0, 2).reshape(
        2 * x.shape[1], kk * W1.shape[-1])
    h1 = _tap_transform(xa_p, w1_2d)
    p1 = _sc_scatter(gE, wE, recv, h1)[:, :n, :]
    xl = _bn_relu(p1, bn1_scale, bn1_offset)

    xl_p = jnp.pad(xl, ((0, np1 - n), (0, 0)))
    w2_2d = W2.reshape(kk, W2.shape[2], -1).transpose(1, 0, 2).reshape(
        W2.shape[2], kk * W2.shape[-1])
    h2 = _tap_transform(xl_p, w2_2d)
    p2 = _sc_scatter(gE, wE, recv, h2)[:, :n, :]
    return _finalize(p2, bn2_scale, bn2_offset, x, y)
